# baseline (speedup 1.0000x reference)
import functools

import jax
import jax.numpy as jnp
from jax import lax
from jax.experimental import pallas as pl
from jax.experimental.pallas import tpu as pltpu
from jax.experimental.pallas import tpu_sc as plsc

N = 10000
E = 320000
G = 8
NP = 10240
NW = 32
EPW = E // NW
AGG_CFG = {64: (128, 6), 128: (128, 6)}
RPT = NP // 16
BN = 512
NB = NP // BN

_SC_MESH = dict(core_axis_name="c", subcore_axis_name="s",
                num_cores=2, num_subcores=16)


def _deg_body(dst_hbm, degp_hbm, dstv, degv):
    w = lax.axis_index("c") * 16 + lax.axis_index("s")
    zeros16 = jnp.zeros((16,), jnp.float32)
    ones16 = jnp.ones((16,), jnp.float32)

    def zero(i, _):
        degv[pl.ds(i * 16, 16)] = zeros16
        return _
    lax.fori_loop(0, NP // 16, zero, None)

    pltpu.sync_copy(dst_hbm.at[w], dstv)

    def body(i, _):
        d16 = dstv[pl.ds(i * 16, 16)]
        plsc.addupdate_scatter(degv, [d16], ones16)
        return _
    lax.fori_loop(0, EPW // 16, body, None)

    pltpu.sync_copy(degv, degp_hbm.at[w])


def _deg_call(dst_flat):
    k = pl.kernel(
        _deg_body,
        out_type=jax.ShapeDtypeStruct((NW, NP), jnp.float32),
        mesh=plsc.VectorSubcoreMesh(**_SC_MESH),
        compiler_params=pltpu.CompilerParams(needs_layout_passes=False),
        scratch_types=[
            pltpu.VMEM((EPW,), jnp.int32),
            pltpu.VMEM((NP,), jnp.float32),
        ],
    )
    return k(dst_flat)


def _agg_body(src_hbm, dst_hbm, v_hbm, out_hbm, srcv, dstv, rows, acc,
              gsem, ssem, *, nch, nbuf):
    cid = lax.axis_index("c")
    sid = lax.axis_index("s")
    w = cid * 16 + sid

    pltpu.sync_copy(v_hbm.at[pl.ds(sid * RPT, RPT)],
                    acc.at[pl.ds(sid * RPT, RPT)])
    plsc.subcore_barrier()

    pltpu.sync_copy(src_hbm.at[w], srcv)
    pltpu.sync_copy(dst_hbm.at[w], dstv)

    for j in range(nbuf - 1):
        pltpu.async_copy(v_hbm.at[srcv.at[j]], rows.at[j], gsem.at[j])

    def body(i, _):
        p = lax.rem(i, nbuf)
        nb = lax.rem(i + nbuf - 1, nbuf)

        @pl.when(i > 0)
        def _():
            pltpu.make_async_copy(rows.at[nb], acc.at[dstv.at[i - 1]],
                                  ssem.at[nb]).wait()

        @pl.when(i + nbuf - 1 < nch)
        def _():
            pltpu.async_copy(v_hbm.at[srcv.at[i + nbuf - 1]], rows.at[nb],
                             gsem.at[nb])

        pltpu.make_async_copy(v_hbm.at[srcv.at[i]], rows.at[p],
                              gsem.at[p]).wait()
        pltpu.async_copy(rows.at[p], acc.at[dstv.at[i]], ssem.at[p],
                         add=True)
        return _
    lax.fori_loop(0, nch, body, None)

    lastp = (nch - 1) % nbuf
    pltpu.make_async_copy(rows.at[lastp], acc.at[dstv.at[nch - 1]],
                          ssem.at[lastp]).wait()
    plsc.subcore_barrier()
    pltpu.sync_copy(acc.at[pl.ds(sid * RPT, RPT)],
                    out_hbm.at[cid, pl.ds(sid * RPT, RPT)])


def _agg_call(src_r, dst_r, v, F):
    ch, nbuf = AGG_CFG[F]
    nch = src_r.shape[1]
    k = pl.kernel(
        functools.partial(_agg_body, nch=nch, nbuf=nbuf),
        out_type=jax.ShapeDtypeStruct((2, NP, F), jnp.bfloat16),
        mesh=plsc.VectorSubcoreMesh(**_SC_MESH),
        compiler_params=pltpu.CompilerParams(use_tc_tiling_on_sc=False),
        scratch_types=[
            pltpu.VMEM((nch, ch), jnp.int32),
            pltpu.VMEM((nch, ch), jnp.int32),
            pltpu.VMEM((nbuf, ch, F), jnp.bfloat16),
            pltpu.VMEM_SHARED((NP, F), jnp.bfloat16),
            pltpu.SemaphoreType.DMA((nbuf,)),
            pltpu.SemaphoreType.DMA((nbuf,)),
        ],
    )
    return k(src_r, dst_r, v)


def _tc_a1(x_ref, w1_ref, h_ref):
    h_ref[...] = jnp.dot(x_ref[...], w1_ref[...],
                         preferred_element_type=jnp.float32)


def _tc_a2(h_ref, degp_ref, v1_ref, dinv_ref):
    deg = jnp.sum(degp_ref[...], axis=0) + 1.0
    dinv = lax.rsqrt(deg)
    v1_ref[...] = (h_ref[...] * dinv[:, None]).astype(jnp.bfloat16)
    dinv_ref[...] = dinv


def _agg_sum(p0_ref, p1_ref, v_ref):
    return (p0_ref[...].astype(jnp.float32) + p1_ref[...].astype(jnp.float32)
            - v_ref[...].astype(jnp.float32))


def _tc_b(p0_ref, p1_ref, v1_ref, dinv_ref, b1_ref, v2_ref):
    dinv = dinv_ref[...][:, None]
    u = dinv * _agg_sum(p0_ref, p1_ref, v1_ref)
    h1 = jnp.maximum(u + b1_ref[...][None, :], 0.0)
    v2_ref[...] = (h1 * dinv).astype(jnp.bfloat16)


def _tc_c(p0_ref, p1_ref, v2_ref, dinv_ref, w2_ref, b2_ref, v3_ref):
    dinv = dinv_ref[...][:, None]
    u = dinv * _agg_sum(p0_ref, p1_ref, v2_ref)
    h2 = jnp.maximum(
        jnp.dot(u, w2_ref[...], preferred_element_type=jnp.float32)
        + b2_ref[...][None, :], 0.0)
    v3_ref[...] = (h2 * dinv).astype(jnp.bfloat16)


def _tc_d(p0_ref, p1_ref, v3_ref, dinv_ref, w3_ref, b3_ref, batch_ref,
          out_ref, sums_ref, cnts_ref):
    i = pl.program_id(0)
    dinv = dinv_ref[...][:, None]
    u = dinv * _agg_sum(p0_ref, p1_ref, v3_ref)
    h3 = jnp.maximum(
        jnp.dot(u, w3_ref[...], preferred_element_type=jnp.float32)
        + b3_ref[...][None, :], 0.0)
    gids = lax.broadcasted_iota(jnp.int32, (G, BN), 0)
    oh = (gids == batch_ref[...][None, :]).astype(jnp.float32)
    psum = jnp.dot(oh, h3, preferred_element_type=jnp.float32)
    pcnt = jnp.sum(oh, axis=1)[:, None]

    @pl.when(i == 0)
    def _():
        sums_ref[...] = psum
        cnts_ref[...] = jnp.broadcast_to(pcnt, (G, 256))

    @pl.when(i > 0)
    def _():
        sums_ref[...] += psum
        cnts_ref[...] += jnp.broadcast_to(pcnt, (G, 256))

    @pl.when(i == pl.num_programs(0) - 1)
    def _():
        out_ref[...] = sums_ref[...] / jnp.maximum(cnts_ref[...], 1.0)


def _row_spec(F):
    return pl.BlockSpec((BN, F), lambda i: (i, 0))


def _full_spec(shape):
    nd = len(shape)
    return pl.BlockSpec(shape, lambda i: (0,) * nd)


_DINV_SPEC = pl.BlockSpec((BN,), lambda i: (i,))


def kernel(x, edge_index, batch, W1, b1, W2, b2, W3, b3):
    f32 = jnp.float32

    def edge_lists(F):
        ch, _ = AGG_CFG[F]
        nch = -(-EPW // ch)
        pad_e = NW * nch * ch - E
        r = jnp.arange(pad_e, dtype=jnp.int32)
        src_p = jnp.concatenate([edge_index[0], r % N])
        dst_p = jnp.concatenate([edge_index[1], N + r % (NP - N)])
        return src_p.reshape(NW, nch, ch), dst_p.reshape(NW, nch, ch)

    src64, dst64 = edge_lists(64)
    src128, dst128 = edge_lists(128)
    dst_flat = edge_index[1].reshape(NW, EPW)
    xp = jnp.pad(x, ((0, NP - N), (0, 0)))
    batchp = jnp.pad(batch, (0, NP - N), constant_values=G)

    degp = _deg_call(dst_flat)

    h1p = pl.pallas_call(
        _tc_a1,
        grid=(NB,),
        in_specs=[_row_spec(128), _full_spec((128, 64))],
        out_specs=_row_spec(64),
        out_shape=jax.ShapeDtypeStruct((NP, 64), f32),
    )(xp, W1)

    v1, dinv = pl.pallas_call(
        _tc_a2,
        grid=(NB,),
        in_specs=[_row_spec(64), pl.BlockSpec((NW, BN), lambda i: (0, i))],
        out_specs=[_row_spec(64), _DINV_SPEC],
        out_shape=[jax.ShapeDtypeStruct((NP, 64), jnp.bfloat16),
                   jax.ShapeDtypeStruct((NP,), f32)],
    )(h1p, degp)

    e1 = _agg_call(src64, dst64, v1, 64)

    v2 = pl.pallas_call(
        _tc_b,
        grid=(NB,),
        in_specs=[_row_spec(64), _row_spec(64), _row_spec(64), _DINV_SPEC,
                  _full_spec((64,))],
        out_specs=_row_spec(64),
        out_shape=jax.ShapeDtypeStruct((NP, 64), jnp.bfloat16),
    )(e1[0], e1[1], v1, dinv, b1)

    e2 = _agg_call(src64, dst64, v2, 64)

    v3 = pl.pallas_call(
        _tc_c,
        grid=(NB,),
        in_specs=[_row_spec(64), _row_spec(64), _row_spec(64), _DINV_SPEC,
                  _full_spec((64, 128)), _full_spec((128,))],
        out_specs=_row_spec(128),
        out_shape=jax.ShapeDtypeStruct((NP, 128), jnp.bfloat16),
    )(e2[0], e2[1], v2, dinv, W2, b2)

    e3 = _agg_call(src128, dst128, v3, 128)

    pooled = pl.pallas_call(
        _tc_d,
        grid=(NB,),
        in_specs=[_row_spec(128), _row_spec(128), _row_spec(128), _DINV_SPEC,
                  _full_spec((128, 256)), _full_spec((256,)),
                  pl.BlockSpec((BN,), lambda i: (i,))],
        out_specs=_full_spec((G, 256)),
        out_shape=jax.ShapeDtypeStruct((G, 256), f32),
        scratch_shapes=[pltpu.VMEM((G, 256), f32), pltpu.VMEM((G, 256), f32)],
    )(e3[0], e3[1], v3, dinv, W3, b3, batchp)

    return pooled


# speedup vs baseline: 1.1222x; 1.1222x over previous
import functools

import jax
import jax.numpy as jnp
from jax import lax
from jax.experimental import pallas as pl
from jax.experimental.pallas import tpu as pltpu
from jax.experimental.pallas import tpu_sc as plsc

N = 10000
E = 320000
G = 8
NP = 10240
NW = 32
EPW = E // NW
AGG_CFG = {64: (128, 8), 128: (128, 8)}
RPT = NP // 16
BN = 1024
NB = NP // BN

_SC_MESH = dict(core_axis_name="c", subcore_axis_name="s",
                num_cores=2, num_subcores=16)


def _deg_body(dst_hbm, degp_hbm, dstv, degv):
    w = lax.axis_index("c") * 16 + lax.axis_index("s")
    zeros16 = jnp.zeros((16,), jnp.float32)
    ones16 = jnp.ones((16,), jnp.float32)

    def zero(i, _):
        degv[pl.ds(i * 16, 16)] = zeros16
        return _
    lax.fori_loop(0, NP // 16, zero, None)

    pltpu.sync_copy(dst_hbm.at[w], dstv)

    def body(i, _):
        d16 = dstv[pl.ds(i * 16, 16)]
        plsc.addupdate_scatter(degv, [d16], ones16)
        return _
    lax.fori_loop(0, EPW // 16, body, None)

    pltpu.sync_copy(degv, degp_hbm.at[w])


def _deg_call(dst_flat):
    k = pl.kernel(
        _deg_body,
        out_type=jax.ShapeDtypeStruct((NW, NP), jnp.float32),
        mesh=plsc.VectorSubcoreMesh(**_SC_MESH),
        compiler_params=pltpu.CompilerParams(needs_layout_passes=False),
        scratch_types=[
            pltpu.VMEM((EPW,), jnp.int32),
            pltpu.VMEM((NP,), jnp.float32),
        ],
    )
    return k(dst_flat)


def _agg_body(src_hbm, dst_hbm, v_hbm, out_hbm, srcv, dstv, rows, acc,
              gsem, ssem, *, nch, nbuf):
    cid = lax.axis_index("c")
    sid = lax.axis_index("s")
    w = cid * 16 + sid

    pltpu.sync_copy(v_hbm.at[pl.ds(sid * RPT, RPT)],
                    acc.at[pl.ds(sid * RPT, RPT)])
    plsc.subcore_barrier()

    pltpu.sync_copy(src_hbm.at[w], srcv)
    pltpu.sync_copy(dst_hbm.at[w], dstv)

    for j in range(nbuf - 1):
        pltpu.async_copy(v_hbm.at[srcv.at[j]], rows.at[j], gsem.at[j])

    def body(i, _):
        p = lax.rem(i, nbuf)
        nb = lax.rem(i + nbuf - 1, nbuf)

        @pl.when(i > 0)
        def _():
            pltpu.make_async_copy(rows.at[nb], acc.at[dstv.at[i - 1]],
                                  ssem.at[nb]).wait()

        @pl.when(i + nbuf - 1 < nch)
        def _():
            pltpu.async_copy(v_hbm.at[srcv.at[i + nbuf - 1]], rows.at[nb],
                             gsem.at[nb])

        pltpu.make_async_copy(v_hbm.at[srcv.at[i]], rows.at[p],
                              gsem.at[p]).wait()
        pltpu.async_copy(rows.at[p], acc.at[dstv.at[i]], ssem.at[p],
                         add=True)
        return _
    lax.fori_loop(0, nch, body, None)

    lastp = (nch - 1) % nbuf
    pltpu.make_async_copy(rows.at[lastp], acc.at[dstv.at[nch - 1]],
                          ssem.at[lastp]).wait()
    plsc.subcore_barrier()
    pltpu.sync_copy(acc.at[pl.ds(sid * RPT, RPT)],
                    out_hbm.at[cid, pl.ds(sid * RPT, RPT)])


def _agg_call(src_r, dst_r, v, F):
    ch, nbuf = AGG_CFG[F]
    nch = src_r.shape[1]
    k = pl.kernel(
        functools.partial(_agg_body, nch=nch, nbuf=nbuf),
        out_type=jax.ShapeDtypeStruct((2, NP, F), jnp.bfloat16),
        mesh=plsc.VectorSubcoreMesh(**_SC_MESH),
        compiler_params=pltpu.CompilerParams(use_tc_tiling_on_sc=False),
        scratch_types=[
            pltpu.VMEM((nch, ch), jnp.int32),
            pltpu.VMEM((nch, ch), jnp.int32),
            pltpu.VMEM((nbuf, ch, F), jnp.bfloat16),
            pltpu.VMEM_SHARED((NP, F), jnp.bfloat16),
            pltpu.SemaphoreType.DMA((nbuf,)),
            pltpu.SemaphoreType.DMA((nbuf,)),
        ],
    )
    return k(src_r, dst_r, v)


def _tc_a(x_ref, w1_ref, degp_ref, v1_ref, dinv_ref):
    deg = jnp.sum(degp_ref[...], axis=0) + 1.0
    dinv = lax.rsqrt(deg)
    h = jnp.dot(x_ref[...], w1_ref[...], preferred_element_type=jnp.float32)
    v1_ref[...] = (h * dinv[:, None]).astype(jnp.bfloat16)
    dinv_ref[...] = dinv


def _agg_sum(p0_ref, p1_ref, v_ref):
    return (p0_ref[...].astype(jnp.float32) + p1_ref[...].astype(jnp.float32)
            - v_ref[...].astype(jnp.float32))


def _tc_b(p0_ref, p1_ref, v1_ref, dinv_ref, b1_ref, v2_ref):
    dinv = dinv_ref[...][:, None]
    u = dinv * _agg_sum(p0_ref, p1_ref, v1_ref)
    h1 = jnp.maximum(u + b1_ref[...][None, :], 0.0)
    v2_ref[...] = (h1 * dinv).astype(jnp.bfloat16)


def _tc_c(p0_ref, p1_ref, v2_ref, dinv_ref, w2_ref, b2_ref, v3_ref):
    dinv = dinv_ref[...][:, None]
    u = dinv * _agg_sum(p0_ref, p1_ref, v2_ref)
    h2 = jnp.maximum(
        jnp.dot(u, w2_ref[...], preferred_element_type=jnp.float32)
        + b2_ref[...][None, :], 0.0)
    v3_ref[...] = (h2 * dinv).astype(jnp.bfloat16)


def _tc_d(p0_ref, p1_ref, v3_ref, dinv_ref, w3_ref, b3_ref, batch_ref,
          out_ref, sums_ref, cnts_ref):
    i = pl.program_id(0)
    dinv = dinv_ref[...][:, None]
    u = dinv * _agg_sum(p0_ref, p1_ref, v3_ref)
    h3 = jnp.maximum(
        jnp.dot(u, w3_ref[...], preferred_element_type=jnp.float32)
        + b3_ref[...][None, :], 0.0)
    gids = lax.broadcasted_iota(jnp.int32, (G, BN), 0)
    oh = (gids == batch_ref[...][None, :]).astype(jnp.float32)
    psum = jnp.dot(oh, h3, preferred_element_type=jnp.float32)
    pcnt = jnp.sum(oh, axis=1)[:, None]

    @pl.when(i == 0)
    def _():
        sums_ref[...] = psum
        cnts_ref[...] = jnp.broadcast_to(pcnt, (G, 256))

    @pl.when(i > 0)
    def _():
        sums_ref[...] += psum
        cnts_ref[...] += jnp.broadcast_to(pcnt, (G, 256))

    @pl.when(i == pl.num_programs(0) - 1)
    def _():
        out_ref[...] = sums_ref[...] / jnp.maximum(cnts_ref[...], 1.0)


def _row_spec(F):
    return pl.BlockSpec((BN, F), lambda i: (i, 0))


def _full_spec(shape):
    nd = len(shape)
    return pl.BlockSpec(shape, lambda i: (0,) * nd)


_DINV_SPEC = pl.BlockSpec((BN,), lambda i: (i,))


def kernel(x, edge_index, batch, W1, b1, W2, b2, W3, b3):
    f32 = jnp.float32

    def edge_lists(F):
        ch, _ = AGG_CFG[F]
        nch = -(-EPW // ch)
        pad_e = NW * nch * ch - E
        r = jnp.arange(pad_e, dtype=jnp.int32)
        src_p = jnp.concatenate([edge_index[0], r % N])
        dst_p = jnp.concatenate([edge_index[1], N + r % (NP - N)])
        return src_p.reshape(NW, nch, ch), dst_p.reshape(NW, nch, ch)

    src64, dst64 = edge_lists(64)
    src128, dst128 = edge_lists(128)
    dst_flat = edge_index[1].reshape(NW, EPW)
    xp = jnp.pad(x, ((0, NP - N), (0, 0)))
    batchp = jnp.pad(batch, (0, NP - N), constant_values=G)

    degp = _deg_call(dst_flat)

    v1, dinv = pl.pallas_call(
        _tc_a,
        grid=(NB,),
        in_specs=[_row_spec(128), _full_spec((128, 64)),
                  pl.BlockSpec((NW, BN), lambda i: (0, i))],
        out_specs=[_row_spec(64), _DINV_SPEC],
        out_shape=[jax.ShapeDtypeStruct((NP, 64), jnp.bfloat16),
                   jax.ShapeDtypeStruct((NP,), f32)],
    )(xp, W1, degp)

    e1 = _agg_call(src64, dst64, v1, 64)

    v2 = pl.pallas_call(
        _tc_b,
        grid=(NB,),
        in_specs=[_row_spec(64), _row_spec(64), _row_spec(64), _DINV_SPEC,
                  _full_spec((64,))],
        out_specs=_row_spec(64),
        out_shape=jax.ShapeDtypeStruct((NP, 64), jnp.bfloat16),
    )(e1[0], e1[1], v1, dinv, b1)

    e2 = _agg_call(src64, dst64, v2, 64)

    v3 = pl.pallas_call(
        _tc_c,
        grid=(NB,),
        in_specs=[_row_spec(64), _row_spec(64), _row_spec(64), _DINV_SPEC,
                  _full_spec((64, 128)), _full_spec((128,))],
        out_specs=_row_spec(128),
        out_shape=jax.ShapeDtypeStruct((NP, 128), jnp.bfloat16),
    )(e2[0], e2[1], v2, dinv, W2, b2)

    e3 = _agg_call(src128, dst128, v3, 128)

    pooled = pl.pallas_call(
        _tc_d,
        grid=(NB,),
        in_specs=[_row_spec(128), _row_spec(128), _row_spec(128), _DINV_SPEC,
                  _full_spec((128, 256)), _full_spec((256,)),
                  pl.BlockSpec((BN,), lambda i: (i,))],
        out_specs=_full_spec((G, 256)),
        out_shape=jax.ShapeDtypeStruct((G, 256), f32),
        scratch_shapes=[pltpu.VMEM((G, 256), f32), pltpu.VMEM((G, 256), f32)],
    )(e3[0], e3[1], v3, dinv, W3, b3, batchp)

    return pooled
